# baseline (speedup 1.0000x reference)
import jax
import jax.numpy as jnp
from jax import lax
from jax.experimental import pallas as pl
from jax.experimental.pallas import tpu as pltpu

ROWS = 128
COLS = 100000
RB = 8
CH = 256
NCHUNK = 391
NR = ROWS // RB

_TINY = 1.1754943508222875e-38
_NEG_INF = float("-inf")
_INT_MAX = 2**31 - 1


def _u32(v):
    return jnp.uint32(v)


def _rotl(x, d):
    return lax.shift_left(x, _u32(d)) | lax.shift_right_logical(x, _u32(32 - d))


def _threefry_gumbel(x1):
    k1 = _u32(42)
    k2 = _u32(0x1BD11BDA) ^ k1
    ks = [_u32(0), k1, k2]
    rot = [[13, 15, 26, 6], [17, 29, 16, 24]]
    x0 = x1
    for g in range(5):
        for i, r in enumerate(rot[g % 2]):
            if not (g == 0 and i == 0):
                x0 = x0 + x1
            x1 = _rotl(x1, r)
            x1 = x1 ^ x0
        x0 = x0 + ks[(g + 1) % 3]
        x1 = x1 + ks[(g + 2) % 3] + _u32(g + 1)
    bits = x0 ^ x1
    fbits = lax.shift_right_logical(bits, _u32(9)) | _u32(0x3F800000)
    f = lax.bitcast_convert_type(fbits, jnp.float32) - jnp.float32(1.0)
    u = jnp.maximum(jnp.float32(_TINY), f)
    return -jnp.log(-jnp.log(u))


def _race_kernel(x_ref, draw_ref, logp_ref, samples_ref, prev_ref):
    r = pl.program_id(0)

    @pl.when(r > 0)
    def _write_prev():
        cols = lax.broadcasted_iota(jnp.int32, (RB, COLS), 1)
        samples_ref[...] = jnp.where(
            cols == prev_ref[...], jnp.float32(1.0), jnp.float32(0.0)
        )

    @pl.when(r < NR)
    def _race():
        _race_body(x_ref, draw_ref, logp_ref, prev_ref, r)


def _race_body(x_ref, draw_ref, logp_ref, prev_ref, r):
    lane = lax.broadcasted_iota(jnp.int32, (RB, CH), 1)
    rowoff42 = (r * RB + lax.broadcasted_iota(jnp.int32, (RB, CH), 0)) * COLS + lane + 42

    m_v = jnp.full((RB, CH), _NEG_INF, jnp.float32)
    s_v = jnp.zeros((RB, CH), jnp.float32)
    b_v = jnp.full((RB, CH), _NEG_INF, jnp.float32)
    k_v = jnp.zeros((RB, CH), jnp.int32)

    for k in range(NCHUNK):
        tail = k == NCHUNK - 1
        base = min(k * CH, COLS - CH)
        xc = x_ref[:, pl.ds(base, CH)]
        g = _threefry_gumbel((rowoff42 + base).astype(jnp.uint32))
        if tail:
            valid = lane >= (NCHUNK - 1) * CH - (COLS - CH)
            xm = jnp.where(valid, xc, _NEG_INF)
            score = jnp.where(valid, g + xc, _NEG_INF)
        else:
            xm = xc
            score = g + xc
        upd = score > b_v
        b_v = jnp.where(upd, score, b_v)
        k_v = jnp.where(upd, k, k_v)
        d = xm - m_v
        e = jnp.exp(-jnp.abs(d))
        s_v = jnp.where(d > 0, s_v * e + jnp.float32(1.0), s_v + e)
        m_v = jnp.maximum(m_v, xm)

    m = jnp.max(m_v, axis=1, keepdims=True)
    s = jnp.sum(s_v * jnp.exp(m_v - m), axis=1, keepdims=True)
    logz = m + jnp.log(s)
    best = jnp.max(b_v, axis=1, keepdims=True)
    eq = b_v == best
    cols_v = k_v * CH + lane
    tail_shift = (NCHUNK - 1) * CH - (COLS - CH)
    cols_v = jnp.where(k_v == NCHUNK - 1, cols_v - tail_shift, cols_v)
    idx = jnp.min(jnp.where(eq, cols_v, _INT_MAX), axis=1, keepdims=True)
    row_ids = r * RB + lax.broadcasted_iota(jnp.int32, (RB, 1), 0)
    g_at = _threefry_gumbel((row_ids * COLS + idx + 42).astype(jnp.uint32))
    logp_ref[...] = (best - g_at) - logz
    draw_ref[...] = idx
    prev_ref[...] = idx


@jax.jit
def kernel(logits):
    draw, logp, samples = pl.pallas_call(
        _race_kernel,
        grid=(NR + 1,),
        in_specs=[
            pl.BlockSpec((RB, COLS), lambda r: (jnp.minimum(r, NR - 1), 0))
        ],
        out_specs=[
            pl.BlockSpec((RB, 1), lambda r: (jnp.minimum(r, NR - 1), 0)),
            pl.BlockSpec((RB, 1), lambda r: (jnp.minimum(r, NR - 1), 0)),
            pl.BlockSpec((RB, COLS), lambda r: (jnp.maximum(r - 1, 0), 0)),
        ],
        out_shape=[
            jax.ShapeDtypeStruct((ROWS, 1), jnp.int32),
            jax.ShapeDtypeStruct((ROWS, 1), jnp.float32),
            jax.ShapeDtypeStruct((ROWS, COLS), jnp.float32),
        ],
        scratch_shapes=[pltpu.VMEM((RB, 1), jnp.int32)],
    )(logits)

    return samples, logp.reshape(ROWS)


# speedup vs baseline: 1.0140x; 1.0140x over previous
import jax
import jax.numpy as jnp
from jax import lax
from jax.experimental import pallas as pl
from jax.experimental.pallas import tpu as pltpu

ROWS = 128
COLS = 100000
RB = 8
CH = 256
NCHUNK = 391
NR = ROWS // RB

_TINY = 1.1754943508222875e-38
_NEG_INF = float("-inf")
_INT_MAX = 2**31 - 1


def _u32(v):
    return jnp.uint32(v)


def _rotl(x, d):
    return lax.shift_left(x, _u32(d)) | lax.shift_right_logical(x, _u32(32 - d))


def _threefry_gumbel(x1):
    k1 = _u32(42)
    k2 = _u32(0x1BD11BDA) ^ k1
    ks = [_u32(0), k1, k2]
    rot = [[13, 15, 26, 6], [17, 29, 16, 24]]
    x0 = x1
    for g in range(5):
        for i, r in enumerate(rot[g % 2]):
            if not (g == 0 and i == 0):
                x0 = x0 + x1
            x1 = _rotl(x1, r)
            x1 = x1 ^ x0
        x0 = x0 + ks[(g + 1) % 3]
        x1 = x1 + ks[(g + 2) % 3] + _u32(g + 1)
    bits = x0 ^ x1
    fbits = lax.shift_right_logical(bits, _u32(9)) | _u32(0x3F800000)
    f = lax.bitcast_convert_type(fbits, jnp.float32) - jnp.float32(1.0)
    u = jnp.maximum(jnp.float32(_TINY), f)
    return -jnp.log(-jnp.log(u))


def _race_kernel(x_ref, draw_ref, logp_ref, samples_ref, prev_ref):
    r = pl.program_id(0)

    @pl.when(r > 0)
    def _write_prev():
        cols = lax.broadcasted_iota(jnp.int32, (RB, COLS), 1)
        samples_ref[...] = jnp.where(
            cols == prev_ref[...], jnp.float32(1.0), jnp.float32(0.0)
        )

    @pl.when(r < NR)
    def _race():
        _race_body(x_ref, draw_ref, logp_ref, prev_ref, r)


def _race_body(x_ref, draw_ref, logp_ref, prev_ref, r):
    lane = lax.broadcasted_iota(jnp.int32, (RB, CH), 1)
    rowoff42 = (r * RB + lax.broadcasted_iota(jnp.int32, (RB, CH), 0)) * COLS + lane + 42

    s_v = jnp.zeros((RB, CH), jnp.float32)
    b_v = jnp.full((RB, CH), _NEG_INF, jnp.float32)
    k_v = jnp.zeros((RB, CH), jnp.int32)

    for k in range(NCHUNK):
        tail = k == NCHUNK - 1
        base = min(k * CH, COLS - CH)
        xc = x_ref[:, pl.ds(base, CH)]
        g = _threefry_gumbel((rowoff42 + base).astype(jnp.uint32))
        if tail:
            valid = lane >= (NCHUNK - 1) * CH - (COLS - CH)
            xm = jnp.where(valid, xc, _NEG_INF)
            score = jnp.where(valid, g + xc, _NEG_INF)
        else:
            xm = xc
            score = g + xc
        upd = score > b_v
        k_v = jnp.where(upd, k, k_v)
        b_new = jnp.maximum(b_v, score)
        s_v = s_v * jnp.exp(b_v - b_new) + jnp.exp(xm - b_new)
        b_v = b_new

    best = jnp.max(b_v, axis=1, keepdims=True)
    s = jnp.sum(s_v * jnp.exp(b_v - best), axis=1, keepdims=True)
    logz = best + jnp.log(s)
    eq = b_v == best
    cols_v = k_v * CH + lane
    tail_shift = (NCHUNK - 1) * CH - (COLS - CH)
    cols_v = jnp.where(k_v == NCHUNK - 1, cols_v - tail_shift, cols_v)
    idx = jnp.min(jnp.where(eq, cols_v, _INT_MAX), axis=1, keepdims=True)
    row_ids = r * RB + lax.broadcasted_iota(jnp.int32, (RB, 1), 0)
    g_at = _threefry_gumbel((row_ids * COLS + idx + 42).astype(jnp.uint32))
    logp_ref[...] = (best - g_at) - logz
    draw_ref[...] = idx
    prev_ref[...] = idx


@jax.jit
def kernel(logits):
    draw, logp, samples = pl.pallas_call(
        _race_kernel,
        grid=(NR + 1,),
        in_specs=[
            pl.BlockSpec((RB, COLS), lambda r: (jnp.minimum(r, NR - 1), 0))
        ],
        out_specs=[
            pl.BlockSpec((RB, 1), lambda r: (jnp.minimum(r, NR - 1), 0)),
            pl.BlockSpec((RB, 1), lambda r: (jnp.minimum(r, NR - 1), 0)),
            pl.BlockSpec((RB, COLS), lambda r: (jnp.maximum(r - 1, 0), 0)),
        ],
        out_shape=[
            jax.ShapeDtypeStruct((ROWS, 1), jnp.int32),
            jax.ShapeDtypeStruct((ROWS, 1), jnp.float32),
            jax.ShapeDtypeStruct((ROWS, COLS), jnp.float32),
        ],
        scratch_shapes=[pltpu.VMEM((RB, 1), jnp.int32)],
    )(logits)

    return samples, logp.reshape(ROWS)


# speedup vs baseline: 1.0175x; 1.0035x over previous
import jax
import jax.numpy as jnp
from jax import lax
from jax.experimental import pallas as pl
from jax.experimental.pallas import tpu as pltpu

ROWS = 128
COLS = 100000
RB = 8
CH = 512
NCHUNK = (COLS + CH - 1) // CH
NR = ROWS // RB

_TINY = 1.1754943508222875e-38
_NEG_INF = float("-inf")
_INT_MAX = 2**31 - 1


def _u32(v):
    return jnp.uint32(v)


def _rotl(x, d):
    return lax.shift_left(x, _u32(d)) | lax.shift_right_logical(x, _u32(32 - d))


def _threefry_gumbel(x1):
    k1 = _u32(42)
    k2 = _u32(0x1BD11BDA) ^ k1
    ks = [_u32(0), k1, k2]
    rot = [[13, 15, 26, 6], [17, 29, 16, 24]]
    x0 = x1
    for g in range(5):
        for i, r in enumerate(rot[g % 2]):
            if not (g == 0 and i == 0):
                x0 = x0 + x1
            x1 = _rotl(x1, r)
            x1 = x1 ^ x0
        x0 = x0 + ks[(g + 1) % 3]
        x1 = x1 + ks[(g + 2) % 3] + _u32(g + 1)
    bits = x0 ^ x1
    fbits = lax.shift_right_logical(bits, _u32(9)) | _u32(0x3F800000)
    f = lax.bitcast_convert_type(fbits, jnp.float32) - jnp.float32(1.0)
    u = jnp.maximum(jnp.float32(_TINY), f)
    return -jnp.log(-jnp.log(u))


def _race_kernel(x_ref, draw_ref, logp_ref, samples_ref, prev_ref):
    r = pl.program_id(0)

    @pl.when(r > 0)
    def _write_prev():
        cols = lax.broadcasted_iota(jnp.int32, (RB, COLS), 1)
        samples_ref[...] = jnp.where(
            cols == prev_ref[...], jnp.float32(1.0), jnp.float32(0.0)
        )

    @pl.when(r < NR)
    def _race():
        _race_body(x_ref, draw_ref, logp_ref, prev_ref, r)


def _race_body(x_ref, draw_ref, logp_ref, prev_ref, r):
    lane = lax.broadcasted_iota(jnp.int32, (RB, CH), 1)
    rowoff42 = (r * RB + lax.broadcasted_iota(jnp.int32, (RB, CH), 0)) * COLS + lane + 42

    s_v = jnp.zeros((RB, CH), jnp.float32)
    b_v = jnp.full((RB, CH), _NEG_INF, jnp.float32)
    k_v = jnp.zeros((RB, CH), jnp.int32)

    for k in range(NCHUNK):
        tail = k == NCHUNK - 1
        base = min(k * CH, COLS - CH)
        xc = x_ref[:, pl.ds(base, CH)]
        g = _threefry_gumbel((rowoff42 + base).astype(jnp.uint32))
        if tail:
            valid = lane >= (NCHUNK - 1) * CH - (COLS - CH)
            xm = jnp.where(valid, xc, _NEG_INF)
            score = jnp.where(valid, g + xc, _NEG_INF)
        else:
            xm = xc
            score = g + xc
        upd = score > b_v
        k_v = jnp.where(upd, k, k_v)
        b_new = jnp.maximum(b_v, score)
        s_v = s_v * jnp.exp(b_v - b_new) + jnp.exp(xm - b_new)
        b_v = b_new

    best = jnp.max(b_v, axis=1, keepdims=True)
    s = jnp.sum(s_v * jnp.exp(b_v - best), axis=1, keepdims=True)
    logz = best + jnp.log(s)
    eq = b_v == best
    cols_v = k_v * CH + lane
    tail_shift = (NCHUNK - 1) * CH - (COLS - CH)
    cols_v = jnp.where(k_v == NCHUNK - 1, cols_v - tail_shift, cols_v)
    idx = jnp.min(jnp.where(eq, cols_v, _INT_MAX), axis=1, keepdims=True)
    row_ids = r * RB + lax.broadcasted_iota(jnp.int32, (RB, 1), 0)
    g_at = _threefry_gumbel((row_ids * COLS + idx + 42).astype(jnp.uint32))
    logp_ref[...] = (best - g_at) - logz
    draw_ref[...] = idx
    prev_ref[...] = idx


@jax.jit
def kernel(logits):
    draw, logp, samples = pl.pallas_call(
        _race_kernel,
        grid=(NR + 1,),
        in_specs=[
            pl.BlockSpec((RB, COLS), lambda r: (jnp.minimum(r, NR - 1), 0))
        ],
        out_specs=[
            pl.BlockSpec((RB, 1), lambda r: (jnp.minimum(r, NR - 1), 0)),
            pl.BlockSpec((RB, 1), lambda r: (jnp.minimum(r, NR - 1), 0)),
            pl.BlockSpec((RB, COLS), lambda r: (jnp.maximum(r - 1, 0), 0)),
        ],
        out_shape=[
            jax.ShapeDtypeStruct((ROWS, 1), jnp.int32),
            jax.ShapeDtypeStruct((ROWS, 1), jnp.float32),
            jax.ShapeDtypeStruct((ROWS, COLS), jnp.float32),
        ],
        scratch_shapes=[pltpu.VMEM((RB, 1), jnp.int32)],
    )(logits)

    return samples, logp.reshape(ROWS)
